# baseline (speedup 1.0000x reference)
import functools

import jax
import jax.numpy as jnp
from jax import lax
from jax.experimental import pallas as pl
from jax.experimental.pallas import tpu as pltpu
from jax.experimental.pallas import tpu_sc as plsc

_BATCH = 16384
_D = 1024
_REP = 32
_STREAM_ROWS = 128


def _make_sc_broadcast(batch, d, rep, stream_rows):
    info = plsc.get_sparse_core_info()
    nc, ns = info.num_cores, info.num_subcores
    nw = nc * ns
    rows_per_w = batch // nw
    dma_rows = rows_per_w - stream_rows
    mesh = plsc.VectorSubcoreMesh(core_axis_name="c", subcore_axis_name="s")

    @functools.partial(
        pl.kernel,
        mesh=mesh,
        out_type=jax.ShapeDtypeStruct((batch, d), jnp.float32),
        scratch_types=[
            pltpu.VMEM((rep,), jnp.int32),
            pltpu.VMEM((rep, d), jnp.float32),
            pltpu.VMEM_SHARED((ns * rep, d), jnp.float32),
            pltpu.SemaphoreType.DMA,
            pltpu.SemaphoreType.DMA,
            pltpu.SemaphoreType.DMA,
        ],
    )
    def sc_broadcast(idx_hbm, table_hbm, out_hbm, idx_v, block_v, shared_v,
                     gsem, ssem, tsem):
        s = lax.axis_index("s")
        wid = s * nc + lax.axis_index("c")
        base = wid * rows_per_w
        pltpu.sync_copy(idx_hbm, idx_v)
        pltpu.async_copy(table_hbm.at[idx_v], block_v, gsem).wait()
        pltpu.sync_copy(block_v, shared_v.at[pl.ds(s * rep, rep)])
        plsc.subcore_barrier()
        tail = pltpu.async_copy(
            shared_v.at[pl.ds(0, dma_rows)],
            out_hbm.at[pl.ds(base + stream_rows, dma_rows)], ssem)
        stream_copies = []
        for j in range(stream_rows // rep):
            stream_copies.append(
                pltpu.async_copy(
                    block_v, out_hbm.at[pl.ds(base + j * rep, rep)], tsem
                )
            )
        for c in stream_copies:
            c.wait()
        tail.wait()

    return sc_broadcast


_sc_broadcast = _make_sc_broadcast(_BATCH, _D, _REP, _STREAM_ROWS)


def kernel(x, lang_type_ids, table):
    del x
    idx = jnp.full((_REP,), lang_type_ids, dtype=jnp.int32)
    return _sc_broadcast(idx, table)


# speedup vs baseline: 1.0598x; 1.0598x over previous
import functools

import jax
import jax.numpy as jnp
from jax import lax
from jax.experimental import pallas as pl
from jax.experimental.pallas import tpu as pltpu
from jax.experimental.pallas import tpu_sc as plsc

_BATCH = 16384
_D = 1024
_REP = 32
_STREAM_ROWS = 224


def _make_sc_broadcast(batch, d, rep, stream_rows):
    info = plsc.get_sparse_core_info()
    nc, ns = info.num_cores, info.num_subcores
    nw = nc * ns
    rows_per_w = batch // nw
    dma_rows = rows_per_w - stream_rows
    mesh = plsc.VectorSubcoreMesh(core_axis_name="c", subcore_axis_name="s")

    @functools.partial(
        pl.kernel,
        mesh=mesh,
        out_type=jax.ShapeDtypeStruct((batch, d), jnp.float32),
        scratch_types=[
            pltpu.VMEM((rep,), jnp.int32),
            pltpu.VMEM((rep, d), jnp.float32),
            pltpu.VMEM_SHARED((ns * rep, d), jnp.float32),
            pltpu.SemaphoreType.DMA,
            pltpu.SemaphoreType.DMA,
            pltpu.SemaphoreType.DMA,
        ],
    )
    def sc_broadcast(idx_hbm, table_hbm, out_hbm, idx_v, block_v, shared_v,
                     gsem, ssem, tsem):
        s = lax.axis_index("s")
        wid = s * nc + lax.axis_index("c")
        base = wid * rows_per_w
        pltpu.sync_copy(idx_hbm, idx_v)
        pltpu.async_copy(table_hbm.at[idx_v], block_v, gsem).wait()
        pltpu.sync_copy(block_v, shared_v.at[pl.ds(s * rep, rep)])
        plsc.subcore_barrier()
        tail = pltpu.async_copy(
            shared_v.at[pl.ds(0, dma_rows)],
            out_hbm.at[pl.ds(base + stream_rows, dma_rows)], ssem)
        stream_copies = []
        for j in range(stream_rows // rep):
            stream_copies.append(
                pltpu.async_copy(
                    block_v, out_hbm.at[pl.ds(base + j * rep, rep)], tsem
                )
            )
        for c in stream_copies:
            c.wait()
        tail.wait()

    return sc_broadcast


_sc_broadcast = _make_sc_broadcast(_BATCH, _D, _REP, _STREAM_ROWS)


def kernel(x, lang_type_ids, table):
    del x
    idx = jnp.full((_REP,), lang_type_ids, dtype=jnp.int32)
    return _sc_broadcast(idx, table)


# speedup vs baseline: 1.0785x; 1.0177x over previous
import functools

import jax
import jax.numpy as jnp
from jax import lax
from jax.experimental import pallas as pl
from jax.experimental.pallas import tpu as pltpu
from jax.experimental.pallas import tpu_sc as plsc

_BATCH = 16384
_D = 1024
_REP = 32
_STREAM_ROWS = 256


def _make_sc_broadcast(batch, d, rep, stream_rows):
    info = plsc.get_sparse_core_info()
    nc, ns = info.num_cores, info.num_subcores
    nw = nc * ns
    rows_per_w = batch // nw
    dma_rows = rows_per_w - stream_rows
    mesh = plsc.VectorSubcoreMesh(core_axis_name="c", subcore_axis_name="s")

    @functools.partial(
        pl.kernel,
        mesh=mesh,
        out_type=jax.ShapeDtypeStruct((batch, d), jnp.float32),
        scratch_types=[
            pltpu.VMEM((rep,), jnp.int32),
            pltpu.VMEM((rep, d), jnp.float32),
            pltpu.VMEM_SHARED((ns * rep, d), jnp.float32),
            pltpu.SemaphoreType.DMA,
            pltpu.SemaphoreType.DMA,
            pltpu.SemaphoreType.DMA,
        ],
    )
    def sc_broadcast(idx_hbm, table_hbm, out_hbm, idx_v, block_v, shared_v,
                     gsem, ssem, tsem):
        s = lax.axis_index("s")
        wid = s * nc + lax.axis_index("c")
        base = wid * rows_per_w
        pltpu.sync_copy(idx_hbm, idx_v)
        pltpu.async_copy(table_hbm.at[idx_v], block_v, gsem).wait()
        pltpu.sync_copy(block_v, shared_v.at[pl.ds(s * rep, rep)])
        plsc.subcore_barrier()
        tail = pltpu.async_copy(
            shared_v.at[pl.ds(0, dma_rows)],
            out_hbm.at[pl.ds(base + stream_rows, dma_rows)], ssem)
        stream_copies = []
        for j in range(stream_rows // rep):
            stream_copies.append(
                pltpu.async_copy(
                    block_v, out_hbm.at[pl.ds(base + j * rep, rep)], tsem
                )
            )
        for c in stream_copies:
            c.wait()
        tail.wait()

    return sc_broadcast


_sc_broadcast = _make_sc_broadcast(_BATCH, _D, _REP, _STREAM_ROWS)


def kernel(x, lang_type_ids, table):
    del x
    idx = jnp.full((_REP,), lang_type_ids, dtype=jnp.int32)
    return _sc_broadcast(idx, table)


# speedup vs baseline: 1.0940x; 1.0144x over previous
import functools

import jax
import jax.numpy as jnp
from jax import lax
from jax.experimental import pallas as pl
from jax.experimental.pallas import tpu as pltpu
from jax.experimental.pallas import tpu_sc as plsc

_BATCH = 16384
_D = 1024
_REP = 32
_STREAM_ROWS = 320


def _make_sc_broadcast(batch, d, rep, stream_rows):
    info = plsc.get_sparse_core_info()
    nc, ns = info.num_cores, info.num_subcores
    nw = nc * ns
    rows_per_w = batch // nw
    dma_rows = rows_per_w - stream_rows
    mesh = plsc.VectorSubcoreMesh(core_axis_name="c", subcore_axis_name="s")

    @functools.partial(
        pl.kernel,
        mesh=mesh,
        out_type=jax.ShapeDtypeStruct((batch, d), jnp.float32),
        scratch_types=[
            pltpu.VMEM((rep,), jnp.int32),
            pltpu.VMEM((rep, d), jnp.float32),
            pltpu.VMEM_SHARED((ns * rep, d), jnp.float32),
            pltpu.SemaphoreType.DMA,
            pltpu.SemaphoreType.DMA,
            pltpu.SemaphoreType.DMA,
        ],
    )
    def sc_broadcast(idx_hbm, table_hbm, out_hbm, idx_v, block_v, shared_v,
                     gsem, ssem, tsem):
        s = lax.axis_index("s")
        wid = s * nc + lax.axis_index("c")
        base = wid * rows_per_w
        pltpu.sync_copy(idx_hbm, idx_v)
        pltpu.async_copy(table_hbm.at[idx_v], block_v, gsem).wait()
        pltpu.sync_copy(block_v, shared_v.at[pl.ds(s * rep, rep)])
        plsc.subcore_barrier()
        tail = pltpu.async_copy(
            shared_v.at[pl.ds(0, dma_rows)],
            out_hbm.at[pl.ds(base + stream_rows, dma_rows)], ssem)
        stream_copies = []
        for j in range(stream_rows // rep):
            stream_copies.append(
                pltpu.async_copy(
                    block_v, out_hbm.at[pl.ds(base + j * rep, rep)], tsem
                )
            )
        for c in stream_copies:
            c.wait()
        tail.wait()

    return sc_broadcast


_sc_broadcast = _make_sc_broadcast(_BATCH, _D, _REP, _STREAM_ROWS)


def kernel(x, lang_type_ids, table):
    del x
    idx = jnp.full((_REP,), lang_type_ids, dtype=jnp.int32)
    return _sc_broadcast(idx, table)
